# Initial kernel scaffold; baseline (speedup 1.0000x reference)
import functools

import jax
import jax.numpy as jnp
from jax import lax
from jax.experimental import pallas as pl
from jax.experimental.pallas import tpu as pltpu
from jax.experimental.pallas import tpu_sc as plsc

N = 10000
E = 320000
D = 128
H = 64
CK = 128
NCH = E // CK
RPT = N // 16
RC = 125
LANES = 16


def _sc_gcn(x2, src, dst, ev):
    mesh = plsc.VectorSubcoreMesh(core_axis_name="c", subcore_axis_name="s")

    @functools.partial(
        pl.kernel,
        out_type=(
            jax.ShapeDtypeStruct((2 * N, H), jnp.float32),
            jax.ShapeDtypeStruct((2 * N, H), jnp.float32),
        ),
        mesh=mesh,
        scratch_types=[
            pltpu.VMEM_SHARED((N, H), jnp.float32),
            pltpu.VMEM((CK,), jnp.int32),
            pltpu.VMEM((CK,), jnp.int32),
            pltpu.VMEM((CK,), jnp.float32),
            pltpu.VMEM((CK,), jnp.int32),
            pltpu.VMEM((CK, H), jnp.float32),
            pltpu.VMEM((RC, H), jnp.float32),
            pltpu.VMEM((RC, H), jnp.float32),
            pltpu.VMEM((RC, H), jnp.float32),
            pltpu.SemaphoreType.DMA,
        ],
    )
    def k(x2_hbm, src_hbm, dst_hbm, ev_hbm, out2_hbm, cur1_hbm,
          acc, dstb, srcb, evb, gidx, rows, fa, fb, fc, sem):
        c = lax.axis_index("c")
        t = lax.axis_index("s")
        coff = c * N
        row0 = t * RPT

        def zero_acc():
            def zb(r, _):
                for kk in range(H // LANES):
                    fa[r, pl.ds(kk * LANES, LANES)] = jnp.zeros(
                        (LANES,), jnp.float32)
                return 0
            lax.fori_loop(0, RC, zb, 0)
            for j in range(RPT // RC):
                pltpu.sync_copy(fa, acc.at[pl.ds(row0 + j * RC, RC)])

        def layer(table_hbm):
            nch = (NCH // 16) + (t < (NCH % 16)).astype(jnp.int32)

            def chunk(i, _):
                base = (t + 16 * i) * CK
                pltpu.sync_copy(dst_hbm.at[pl.ds(base, CK)], dstb)
                pltpu.sync_copy(src_hbm.at[pl.ds(base, CK)], srcb)
                pltpu.sync_copy(ev_hbm.at[pl.ds(base, CK)], evb)
                for j in range(CK // LANES):
                    sl = pl.ds(j * LANES, LANES)
                    gidx[sl] = dstb[sl] + coff
                pltpu.async_copy(table_hbm.at[gidx], rows, sem).wait()

                def escale(e, _):
                    s = plsc.load_gather(
                        evb, [jnp.full((LANES,), e, jnp.int32)])
                    for kk in range(H // LANES):
                        sl = pl.ds(kk * LANES, LANES)
                        rows[e, sl] = rows[e, sl] * s
                    return 0
                lax.fori_loop(0, CK, escale, 0)
                pltpu.sync_copy(rows, acc.at[srcb], add=True)
                return 0
            lax.fori_loop(0, nch, chunk, 0)

        def flush(dst_hbm_buf):
            for j in range(RPT // RC):
                r0 = row0 + j * RC
                pltpu.sync_copy(acc.at[pl.ds(r0, RC)], fa)
                pltpu.sync_copy(fa, dst_hbm_buf.at[pl.ds(coff + r0, RC)])

        zero_acc()
        plsc.subcore_barrier()
        layer(x2_hbm)
        plsc.subcore_barrier()
        flush(cur1_hbm)
        plsc.subcore_barrier()

        zero_acc()
        plsc.subcore_barrier()
        layer(cur1_hbm)
        plsc.subcore_barrier()

        third = jnp.float32(1.0 / 3.0)
        for j in range(RPT // RC):
            r0 = row0 + j * RC
            pltpu.sync_copy(acc.at[pl.ds(r0, RC)], fa)
            pltpu.sync_copy(x2_hbm.at[pl.ds(coff + r0, RC)], fb)
            pltpu.sync_copy(cur1_hbm.at[pl.ds(coff + r0, RC)], fc)

            def mean_row(r, _):
                for kk in range(H // LANES):
                    sl = pl.ds(kk * LANES, LANES)
                    fa[r, sl] = (fa[r, sl] + fb[r, sl] + fc[r, sl]) * third
                return 0
            lax.fori_loop(0, RC, mean_row, 0)
            pltpu.sync_copy(fa, out2_hbm.at[pl.ds(coff + r0, RC)])

    return k(x2, src, dst, ev)


def kernel(x, edge_index, edge_values, keep_rate):
    del keep_rate
    src = edge_index[0]
    dst = edge_index[1]
    x2 = x.reshape(N, 2, H).transpose(1, 0, 2).reshape(2 * N, H)
    out2, _ = _sc_gcn(x2, src, dst, edge_values)
    return out2.reshape(2, N, H).transpose(1, 0, 2).reshape(N, D)


# speedup vs baseline: 2.2036x; 2.2036x over previous
import functools

import jax
import jax.numpy as jnp
from jax import lax
from jax.experimental import pallas as pl
from jax.experimental.pallas import tpu as pltpu
from jax.experimental.pallas import tpu_sc as plsc

N = 10000
NP = 10240
E = 320000
D = 128
H = 64
CK = 128
NCH = E // CK
RPT = NP // 16
RC = 128
LANES = 16


def _sc_gcn(x2, src, dst, ev):
    mesh = plsc.VectorSubcoreMesh(core_axis_name="c", subcore_axis_name="s")

    @functools.partial(
        pl.kernel,
        out_type=(
            jax.ShapeDtypeStruct((2 * NP, H), jnp.float32),
            jax.ShapeDtypeStruct((2 * NP, H), jnp.float32),
        ),
        mesh=mesh,
        scratch_types=[
            pltpu.VMEM_SHARED((NP, H), jnp.float32),
            pltpu.VMEM((CK,), jnp.int32),
            pltpu.VMEM((CK,), jnp.int32),
            pltpu.VMEM((CK,), jnp.float32),
            pltpu.VMEM((CK,), jnp.int32),
            pltpu.VMEM((CK, H), jnp.float32),
            pltpu.VMEM((RC, H), jnp.float32),
            pltpu.VMEM((RC, H), jnp.float32),
            pltpu.VMEM((RC, H), jnp.float32),
            pltpu.SemaphoreType.DMA,
        ],
        compiler_params=pltpu.CompilerParams(use_tc_tiling_on_sc=False),
    )
    def k(x2_hbm, src_hbm, dst_hbm, ev_hbm, out2_hbm, cur1_hbm,
          acc, dstb, srcb, evb, gidx, rows, fa, fb, fc, sem):
        c = lax.axis_index("c")
        t = lax.axis_index("s")
        coff = c * NP
        row0 = t * RPT

        def zero_acc():
            def zb(r, _):
                for kk in range(H // LANES):
                    fa[r, pl.ds(kk * LANES, LANES)] = jnp.zeros(
                        (LANES,), jnp.float32)
                return 0
            lax.fori_loop(0, RC, zb, 0)
            for j in range(RPT // RC):
                pltpu.sync_copy(fa, acc.at[pl.ds(row0 + j * RC, RC)])

        def layer(table_hbm):
            nch = (NCH // 16) + (t < (NCH % 16)).astype(jnp.int32)

            def chunk(i, _):
                base = (t + 16 * i) * CK
                pltpu.sync_copy(dst_hbm.at[pl.ds(base, CK)], dstb)
                pltpu.sync_copy(src_hbm.at[pl.ds(base, CK)], srcb)
                pltpu.sync_copy(ev_hbm.at[pl.ds(base, CK)], evb)
                for j in range(CK // LANES):
                    sl = pl.ds(j * LANES, LANES)
                    gidx[sl] = dstb[sl] + coff
                pltpu.async_copy(table_hbm.at[gidx], rows, sem).wait()

                def escale(g, _):
                    ev16 = evb[pl.ds(g * LANES, LANES)]
                    for l in range(LANES):
                        s = jnp.full((LANES,), ev16[l], jnp.float32)
                        e = g * LANES + l
                        for kk in range(H // LANES):
                            sl = pl.ds(kk * LANES, LANES)
                            rows[e, sl] = rows[e, sl] * s
                    return 0
                lax.fori_loop(0, CK // LANES, escale, 0)
                pltpu.sync_copy(rows, acc.at[srcb], add=True)
                return 0
            lax.fori_loop(0, nch, chunk, 0)

        def flush(dst_hbm_buf):
            for j in range(RPT // RC):
                r0 = row0 + j * RC
                pltpu.sync_copy(acc.at[pl.ds(r0, RC)], fa)
                pltpu.sync_copy(fa, dst_hbm_buf.at[pl.ds(coff + r0, RC)])

        zero_acc()
        plsc.subcore_barrier()
        layer(x2_hbm)
        plsc.subcore_barrier()
        flush(cur1_hbm)
        plsc.subcore_barrier()

        zero_acc()
        plsc.subcore_barrier()
        layer(cur1_hbm)
        plsc.subcore_barrier()

        third = jnp.float32(1.0 / 3.0)
        for j in range(RPT // RC):
            r0 = row0 + j * RC
            pltpu.sync_copy(acc.at[pl.ds(r0, RC)], fa)
            pltpu.sync_copy(x2_hbm.at[pl.ds(coff + r0, RC)], fb)
            pltpu.sync_copy(cur1_hbm.at[pl.ds(coff + r0, RC)], fc)

            def mean_row(r, _):
                for kk in range(H // LANES):
                    sl = pl.ds(kk * LANES, LANES)
                    fa[r, sl] = (fa[r, sl] + fb[r, sl] + fc[r, sl]) * third
                return 0
            lax.fori_loop(0, RC, mean_row, 0)
            pltpu.sync_copy(fa, out2_hbm.at[pl.ds(coff + r0, RC)])

    return k(x2, src, dst, ev)


def kernel(x, edge_index, edge_values, keep_rate):
    del keep_rate
    src = edge_index[0]
    dst = edge_index[1]
    x2 = x.reshape(N, 2, H).transpose(1, 0, 2)
    x2 = jnp.pad(x2, ((0, 0), (0, NP - N), (0, 0))).reshape(2 * NP, H)
    out2, _ = _sc_gcn(x2, src, dst, edge_values)
    return out2.reshape(2, NP, H)[:, :N].transpose(1, 0, 2).reshape(N, D)


# speedup vs baseline: 2.7916x; 1.2668x over previous
import functools

import jax
import jax.numpy as jnp
from jax import lax
from jax.experimental import pallas as pl
from jax.experimental.pallas import tpu as pltpu
from jax.experimental.pallas import tpu_sc as plsc

N = 10000
NP = 10240
E = 320000
D = 128
H = 64
CK = 128
BLK = 8
NB = 20
EP = 16 * NB * BLK * CK
RPT = NP // 16
RC = 128
LANES = 16


def _sc_gcn(x2, src2, dst2, ev2):
    mesh = plsc.VectorSubcoreMesh(core_axis_name="c", subcore_axis_name="s")

    @functools.partial(
        pl.kernel,
        out_type=(
            jax.ShapeDtypeStruct((2 * NP, H), jnp.float32),
            jax.ShapeDtypeStruct((2 * NP, H), jnp.float32),
        ),
        mesh=mesh,
        scratch_types=[
            pltpu.VMEM_SHARED((NP, H), jnp.float32),
            pltpu.VMEM((BLK, CK), jnp.int32),
            pltpu.VMEM((BLK, CK), jnp.int32),
            pltpu.VMEM((BLK, CK), jnp.float32),
            pltpu.VMEM((BLK, CK), jnp.int32),
            pltpu.VMEM((CK, H), jnp.float32),
            pltpu.VMEM((CK, H), jnp.float32),
            pltpu.VMEM((RC, H), jnp.float32),
            pltpu.VMEM((RC, H), jnp.float32),
            pltpu.VMEM((RC, H), jnp.float32),
            pltpu.SemaphoreType.DMA,
            pltpu.SemaphoreType.DMA,
        ],
        compiler_params=pltpu.CompilerParams(use_tc_tiling_on_sc=False),
    )
    def k(x2_hbm, src_hbm, dst_hbm, ev_hbm, out2_hbm, cur1_hbm,
          acc, dstb, srcb, evb, gidx, rowsA, rowsB, fa, fb, fc,
          gsemA, gsemB):
        c = lax.axis_index("c")
        t = lax.axis_index("s")
        coff = c * NP
        row0 = t * RPT

        def zero_acc():
            def zb(r, _):
                for kk in range(H // LANES):
                    fa[r, pl.ds(kk * LANES, LANES)] = jnp.zeros(
                        (LANES,), jnp.float32)
                return 0
            lax.fori_loop(0, RC, zb, 0)
            for j in range(RPT // RC):
                pltpu.sync_copy(fa, acc.at[pl.ds(row0 + j * RC, RC)])

        def layer(table_hbm):
            def block(i, _):
                brow = (t * NB + i) * BLK
                pltpu.sync_copy(dst_hbm.at[pl.ds(brow, BLK)], dstb)
                pltpu.sync_copy(src_hbm.at[pl.ds(brow, BLK)], srcb)
                pltpu.sync_copy(ev_hbm.at[pl.ds(brow, BLK)], evb)
                for j in range(BLK):
                    for g in range(CK // LANES):
                        sl = pl.ds(g * LANES, LANES)
                        gidx[j, sl] = dstb[j, sl] + coff
                bufs = (rowsA, rowsB)
                sems = (gsemA, gsemB)
                pltpu.async_copy(table_hbm.at[gidx.at[0]], rowsA, gsemA)
                for j in range(BLK):
                    rows, sem = bufs[j % 2], sems[j % 2]
                    if j + 1 < BLK:
                        pltpu.async_copy(
                            table_hbm.at[gidx.at[j + 1]],
                            bufs[(j + 1) % 2], sems[(j + 1) % 2])
                    pltpu.make_async_copy(
                        table_hbm.at[gidx.at[j]], rows, sem).wait()

                    def escale(g, _):
                        ev16 = evb[j, pl.ds(g * LANES, LANES)]
                        for l in range(LANES):
                            s = jnp.full((LANES,), ev16[l], jnp.float32)
                            e = g * LANES + l
                            for kk in range(H // LANES):
                                sl = pl.ds(kk * LANES, LANES)
                                rows[e, sl] = rows[e, sl] * s
                        return 0
                    lax.fori_loop(0, CK // LANES, escale, 0)
                    pltpu.sync_copy(rows, acc.at[srcb.at[j]], add=True)
                return 0
            lax.fori_loop(0, NB, block, 0)

        def flush(dst_hbm_buf):
            for j in range(RPT // RC):
                r0 = row0 + j * RC
                pltpu.sync_copy(acc.at[pl.ds(r0, RC)], fa)
                pltpu.sync_copy(fa, dst_hbm_buf.at[pl.ds(coff + r0, RC)])

        zero_acc()
        plsc.subcore_barrier()
        layer(x2_hbm)
        plsc.subcore_barrier()
        flush(cur1_hbm)
        plsc.subcore_barrier()

        zero_acc()
        plsc.subcore_barrier()
        layer(cur1_hbm)
        plsc.subcore_barrier()

        third = jnp.float32(1.0 / 3.0)
        for j in range(RPT // RC):
            r0 = row0 + j * RC
            pltpu.sync_copy(acc.at[pl.ds(r0, RC)], fa)
            pltpu.sync_copy(x2_hbm.at[pl.ds(coff + r0, RC)], fb)
            pltpu.sync_copy(cur1_hbm.at[pl.ds(coff + r0, RC)], fc)

            def mean_row(r, _):
                for kk in range(H // LANES):
                    sl = pl.ds(kk * LANES, LANES)
                    fa[r, sl] = (fa[r, sl] + fb[r, sl] + fc[r, sl]) * third
                return 0
            lax.fori_loop(0, RC, mean_row, 0)
            pltpu.sync_copy(fa, out2_hbm.at[pl.ds(coff + r0, RC)])

    return k(x2, src2, dst2, ev2)


def kernel(x, edge_index, edge_values, keep_rate):
    del keep_rate
    pad = EP - E
    src = jnp.concatenate([edge_index[0], jnp.zeros((pad,), jnp.int32)])
    dst = jnp.concatenate([edge_index[1], jnp.zeros((pad,), jnp.int32)])
    ev = jnp.concatenate([edge_values, jnp.zeros((pad,), jnp.float32)])
    src2 = src.reshape(EP // CK, CK)
    dst2 = dst.reshape(EP // CK, CK)
    ev2 = ev.reshape(EP // CK, CK)
    x2 = x.reshape(N, 2, H).transpose(1, 0, 2)
    x2 = jnp.pad(x2, ((0, 0), (0, NP - N), (0, 0))).reshape(2 * NP, H)
    out2, _ = _sc_gcn(x2, src2, dst2, ev2)
    return out2.reshape(2, NP, H)[:, :N].transpose(1, 0, 2).reshape(N, D)


# speedup vs baseline: 3.2546x; 1.1658x over previous
import functools

import jax
import jax.numpy as jnp
from jax import lax
from jax.experimental import pallas as pl
from jax.experimental.pallas import tpu as pltpu
from jax.experimental.pallas import tpu_sc as plsc

N = 10000
NP = 10240
E = 320000
D = 128
H = 64
CK = 128
BLK = 8
NB = 20
EP = 16 * NB * BLK * CK
RPT = NP // 16
RC = 128
LANES = 16
GRP = CK // LANES


def _sc_gcn(x2, epack, ev2):
    mesh = plsc.VectorSubcoreMesh(core_axis_name="c", subcore_axis_name="s")

    @functools.partial(
        pl.kernel,
        out_type=(
            jax.ShapeDtypeStruct((2 * NP, H), jnp.float32),
            jax.ShapeDtypeStruct((2 * NP, H), jnp.float32),
        ),
        mesh=mesh,
        scratch_types=[
            pltpu.VMEM_SHARED((NP, H), jnp.float32),
            pltpu.VMEM((BLK, 2, CK), jnp.int32),
            pltpu.VMEM((BLK, 2, CK), jnp.int32),
            pltpu.VMEM((BLK, CK), jnp.float32),
            pltpu.VMEM((BLK, CK), jnp.float32),
            pltpu.VMEM((BLK, CK), jnp.int32),
            pltpu.VMEM((CK, H), jnp.float32),
            pltpu.VMEM((CK, H), jnp.float32),
            pltpu.VMEM((CK, H), jnp.float32),
            pltpu.VMEM((RC, H), jnp.float32),
            pltpu.VMEM((RC, H), jnp.float32),
            pltpu.VMEM((RC, H), jnp.float32),
            pltpu.SemaphoreType.DMA,
            pltpu.SemaphoreType.DMA,
            pltpu.SemaphoreType.DMA,
            pltpu.SemaphoreType.DMA,
            pltpu.SemaphoreType.DMA,
            pltpu.SemaphoreType.DMA,
            pltpu.SemaphoreType.DMA,
            pltpu.SemaphoreType.DMA,
        ],
        compiler_params=pltpu.CompilerParams(use_tc_tiling_on_sc=False),
    )
    def k(x2_hbm, ep_hbm, ev_hbm, out2_hbm, cur1_hbm,
          acc, eA, eB, evA, evB, gidx, r0b, r1b, r2b, fa, fb, fc,
          esA, esB, gs0, gs1, gs2, ss0, ss1, ss2):
        c = lax.axis_index("c")
        t = lax.axis_index("s")
        coff = c * NP
        row0 = t * RPT
        rbufs = (r0b, r1b, r2b)
        gsems = (gs0, gs1, gs2)
        ssems = (ss0, ss1, ss2)

        def zero_acc():
            def zb(r, _):
                for kk in range(H // LANES):
                    fa[r, pl.ds(kk * LANES, LANES)] = jnp.zeros(
                        (LANES,), jnp.float32)
                return 0
            lax.fori_loop(0, RC, zb, 0)
            for j in range(RPT // RC):
                pltpu.sync_copy(fa, acc.at[pl.ds(row0 + j * RC, RC)])

        def layer(table_hbm):
            def start_prefetch(i, eb, evb, es):
                sl = pl.ds((t * NB + i) * BLK, BLK)
                pltpu.async_copy(ep_hbm.at[sl], eb, es)
                pltpu.async_copy(ev_hbm.at[sl], evb, es)

            def wait_prefetch(i, eb, evb, es):
                sl = pl.ds((t * NB + i) * BLK, BLK)
                pltpu.make_async_copy(ep_hbm.at[sl], eb, es).wait()
                pltpu.make_async_copy(ev_hbm.at[sl], evb, es).wait()

            def gather(j, eb):
                pltpu.async_copy(
                    table_hbm.at[gidx.at[j]], rbufs[j % 3], gsems[j % 3])

            def wait_gather(j):
                pltpu.make_async_copy(
                    table_hbm.at[gidx.at[j]], rbufs[j % 3],
                    gsems[j % 3]).wait()

            def scatter(j, eb):
                pltpu.async_copy(
                    rbufs[j % 3], acc.at[eb.at[j, 0]], ssems[j % 3], add=True)

            def wait_scatter(j, eb):
                pltpu.make_async_copy(
                    rbufs[j % 3], acc.at[eb.at[j, 0]], ssems[j % 3]).wait()

            def process_block(eb, evb):
                rows = rbufs
                for j in range(BLK):
                    for g in range(GRP):
                        sl = pl.ds(g * LANES, LANES)
                        gidx[j, sl] = eb[j, 1, sl] + coff
                gather(0, eb)
                gather(1, eb)
                for j in range(BLK):
                    wait_gather(j)
                    rj = rbufs[j % 3]

                    def escale(g, _):
                        ev16 = evb[j, pl.ds(g * LANES, LANES)]
                        for l in range(LANES):
                            s = jnp.full((LANES,), ev16[l], jnp.float32)
                            e = g * LANES + l
                            for kk in range(H // LANES):
                                sl = pl.ds(kk * LANES, LANES)
                                rj[e, sl] = rj[e, sl] * s
                        return 0
                    lax.fori_loop(0, GRP, escale, 0)
                    scatter(j, eb)
                    if j + 2 < BLK:
                        if j - 1 >= 0:
                            wait_scatter(j - 1, eb)
                        gather(j + 2, eb)
                for j in range(BLK - 3, BLK):
                    wait_scatter(j, eb)

            start_prefetch(0, eA, evA, esA)

            def pair(i, _):
                a = 2 * i
                wait_prefetch(a, eA, evA, esA)
                start_prefetch(a + 1, eB, evB, esB)
                process_block(eA, evA)
                wait_prefetch(a + 1, eB, evB, esB)

                @pl.when(i < NB // 2 - 1)
                def _():
                    start_prefetch(a + 2, eA, evA, esA)
                process_block(eB, evB)
                return 0
            lax.fori_loop(0, NB // 2, pair, 0)

        def flush(dst_hbm_buf):
            for j in range(RPT // RC):
                r0 = row0 + j * RC
                pltpu.sync_copy(acc.at[pl.ds(r0, RC)], fa)
                pltpu.sync_copy(fa, dst_hbm_buf.at[pl.ds(coff + r0, RC)])

        zero_acc()
        plsc.subcore_barrier()
        layer(x2_hbm)
        plsc.subcore_barrier()
        flush(cur1_hbm)
        plsc.subcore_barrier()

        zero_acc()
        plsc.subcore_barrier()
        layer(cur1_hbm)
        plsc.subcore_barrier()

        third = jnp.float32(1.0 / 3.0)
        for j in range(RPT // RC):
            r0 = row0 + j * RC
            pltpu.sync_copy(acc.at[pl.ds(r0, RC)], fa)
            pltpu.sync_copy(x2_hbm.at[pl.ds(coff + r0, RC)], fb)
            pltpu.sync_copy(cur1_hbm.at[pl.ds(coff + r0, RC)], fc)

            def mean_row(r, _):
                for kk in range(H // LANES):
                    sl = pl.ds(kk * LANES, LANES)
                    fa[r, sl] = (fa[r, sl] + fb[r, sl] + fc[r, sl]) * third
                return 0
            lax.fori_loop(0, RC, mean_row, 0)
            pltpu.sync_copy(fa, out2_hbm.at[pl.ds(coff + r0, RC)])

    return k(x2, epack, ev2)


def kernel(x, edge_index, edge_values, keep_rate):
    del keep_rate
    pad = EP - E
    src = jnp.concatenate([edge_index[0], jnp.zeros((pad,), jnp.int32)])
    dst = jnp.concatenate([edge_index[1], jnp.zeros((pad,), jnp.int32)])
    ev = jnp.concatenate([edge_values, jnp.zeros((pad,), jnp.float32)])
    epack = jnp.stack(
        [src.reshape(EP // CK, CK), dst.reshape(EP // CK, CK)], axis=1)
    ev2 = ev.reshape(EP // CK, CK)
    x2 = x.reshape(N, 2, H).transpose(1, 0, 2)
    x2 = jnp.pad(x2, ((0, 0), (0, NP - N), (0, 0))).reshape(2 * NP, H)
    out2, _ = _sc_gcn(x2, epack, ev2)
    return out2.reshape(2, NP, H)[:, :N].transpose(1, 0, 2).reshape(N, D)


# speedup vs baseline: 3.2776x; 1.0071x over previous
import functools

import jax
import jax.numpy as jnp
from jax import lax
from jax.experimental import pallas as pl
from jax.experimental.pallas import tpu as pltpu
from jax.experimental.pallas import tpu_sc as plsc

N = 10000
NP = 10240
E = 320000
D = 128
H = 64
CK = 128
BLK = 8
NB = 20
EP = 16 * NB * BLK * CK
RPT = NP // 16
RC = 64
LANES = 16
GRP = CK // LANES


def _sc_gcn(x2, epack, ev2):
    mesh = plsc.VectorSubcoreMesh(core_axis_name="c", subcore_axis_name="s")

    @functools.partial(
        pl.kernel,
        out_type=(
            jax.ShapeDtypeStruct((2 * NP, H), jnp.float32),
            jax.ShapeDtypeStruct((2 * NP, H), jnp.float32),
        ),
        mesh=mesh,
        scratch_types=[
            pltpu.VMEM_SHARED((NP, H), jnp.float32),
            pltpu.VMEM((BLK, 2, CK), jnp.int32),
            pltpu.VMEM((BLK, 2, CK), jnp.int32),
            pltpu.VMEM((BLK, CK), jnp.float32),
            pltpu.VMEM((BLK, CK), jnp.float32),
            pltpu.VMEM((BLK, CK), jnp.int32),
        ] + [pltpu.VMEM((CK, H), jnp.float32)] * BLK + [
            pltpu.VMEM((RC, H), jnp.float32),
            pltpu.VMEM((RC, H), jnp.float32),
            pltpu.VMEM((RC, H), jnp.float32),
            pltpu.SemaphoreType.DMA,
            pltpu.SemaphoreType.DMA,
        ] + [pltpu.SemaphoreType.DMA] * BLK
          + [pltpu.SemaphoreType.DMA] * BLK,
        compiler_params=pltpu.CompilerParams(use_tc_tiling_on_sc=False),
    )
    def k(x2_hbm, ep_hbm, ev_hbm, out2_hbm, cur1_hbm,
          acc, eA, eB, evA, evB, gidx, *rest):
        rbufs = rest[:BLK]
        fa, fb, fc = rest[BLK:BLK + 3]
        esA, esB = rest[BLK + 3:BLK + 5]
        gsems = rest[BLK + 5:2 * BLK + 5]
        ssems = rest[2 * BLK + 5:3 * BLK + 5]
        c = lax.axis_index("c")
        t = lax.axis_index("s")
        coff = c * NP
        row0 = t * RPT

        def zero_acc():
            def zb(r, _):
                for kk in range(H // LANES):
                    fa[r, pl.ds(kk * LANES, LANES)] = jnp.zeros(
                        (LANES,), jnp.float32)
                return 0
            lax.fori_loop(0, RC, zb, 0)
            for j in range(RPT // RC):
                pltpu.sync_copy(fa, acc.at[pl.ds(row0 + j * RC, RC)])

        def layer(table_hbm):
            def start_prefetch(i, eb, evb, es):
                sl = pl.ds((t * NB + i) * BLK, BLK)
                pltpu.async_copy(ep_hbm.at[sl], eb, es)
                pltpu.async_copy(ev_hbm.at[sl], evb, es)

            def wait_prefetch(i, eb, evb, es):
                sl = pl.ds((t * NB + i) * BLK, BLK)
                pltpu.make_async_copy(ep_hbm.at[sl], eb, es).wait()
                pltpu.make_async_copy(ev_hbm.at[sl], evb, es).wait()

            GLA = 4

            def gather(j, eb):
                pltpu.async_copy(
                    table_hbm.at[gidx.at[j]], rbufs[j], gsems[j])

            def wait_gather(j):
                pltpu.make_async_copy(
                    table_hbm.at[gidx.at[j]], rbufs[j], gsems[j]).wait()

            def scatter(j, eb):
                pltpu.async_copy(
                    rbufs[j], acc.at[eb.at[j, 0]], ssems[j], add=True)

            def wait_scatter(j, eb):
                pltpu.make_async_copy(
                    rbufs[j], acc.at[eb.at[j, 0]], ssems[j]).wait()

            def process_block(eb, evb, not_first):
                for j in range(BLK):
                    for g in range(GRP):
                        sl = pl.ds(g * LANES, LANES)
                        gidx[j, sl] = eb[j, 1, sl] + coff
                for j in range(GLA):
                    @pl.when(not_first)
                    def _(j=j):
                        wait_scatter(j, eb)
                    gather(j, eb)
                for j in range(BLK):
                    wait_gather(j)
                    rj = rbufs[j]

                    def escale(g, _):
                        ev16 = evb[j, pl.ds(g * LANES, LANES)]
                        for l in range(LANES):
                            s = jnp.full((LANES,), ev16[l], jnp.float32)
                            e = g * LANES + l
                            for kk in range(H // LANES):
                                sl = pl.ds(kk * LANES, LANES)
                                rj[e, sl] = rj[e, sl] * s
                        return 0
                    lax.fori_loop(0, GRP, escale, 0)
                    scatter(j, eb)
                    if j + GLA < BLK:
                        @pl.when(not_first)
                        def _(j=j):
                            wait_scatter(j + GLA, eb)
                        gather(j + GLA, eb)

            start_prefetch(0, eA, evA, esA)

            def pair(i, _):
                a = 2 * i
                wait_prefetch(a, eA, evA, esA)
                start_prefetch(a + 1, eB, evB, esB)
                process_block(eA, evA, i > 0)
                wait_prefetch(a + 1, eB, evB, esB)

                @pl.when(i < NB // 2 - 1)
                def _():
                    start_prefetch(a + 2, eA, evA, esA)
                process_block(eB, evB, i >= 0)
                return 0
            lax.fori_loop(0, NB // 2, pair, 0)
            for j in range(BLK):
                wait_scatter(j, eB)

        def flush(dst_hbm_buf):
            for j in range(RPT // RC):
                r0 = row0 + j * RC
                pltpu.sync_copy(acc.at[pl.ds(r0, RC)], fa)
                pltpu.sync_copy(fa, dst_hbm_buf.at[pl.ds(coff + r0, RC)])

        zero_acc()
        plsc.subcore_barrier()
        layer(x2_hbm)
        plsc.subcore_barrier()
        flush(cur1_hbm)
        plsc.subcore_barrier()

        zero_acc()
        plsc.subcore_barrier()
        layer(cur1_hbm)
        plsc.subcore_barrier()

        third = jnp.float32(1.0 / 3.0)
        for j in range(RPT // RC):
            r0 = row0 + j * RC
            pltpu.sync_copy(acc.at[pl.ds(r0, RC)], fa)
            pltpu.sync_copy(x2_hbm.at[pl.ds(coff + r0, RC)], fb)
            pltpu.sync_copy(cur1_hbm.at[pl.ds(coff + r0, RC)], fc)

            def mean_row(r, _):
                for kk in range(H // LANES):
                    sl = pl.ds(kk * LANES, LANES)
                    fa[r, sl] = (fa[r, sl] + fb[r, sl] + fc[r, sl]) * third
                return 0
            lax.fori_loop(0, RC, mean_row, 0)
            pltpu.sync_copy(fa, out2_hbm.at[pl.ds(coff + r0, RC)])

    return k(x2, epack, ev2)


def kernel(x, edge_index, edge_values, keep_rate):
    del keep_rate
    pad = EP - E
    src = jnp.concatenate([edge_index[0], jnp.zeros((pad,), jnp.int32)])
    dst = jnp.concatenate([edge_index[1], jnp.zeros((pad,), jnp.int32)])
    ev = jnp.concatenate([edge_values, jnp.zeros((pad,), jnp.float32)])
    epack = jnp.stack(
        [src.reshape(EP // CK, CK), dst.reshape(EP // CK, CK)], axis=1)
    ev2 = ev.reshape(EP // CK, CK)
    x2 = x.reshape(N, 2, H).transpose(1, 0, 2)
    x2 = jnp.pad(x2, ((0, 0), (0, NP - N), (0, 0))).reshape(2 * NP, H)
    out2, _ = _sc_gcn(x2, epack, ev2)
    return out2.reshape(2, NP, H)[:, :N].transpose(1, 0, 2).reshape(N, D)


# speedup vs baseline: 4.6547x; 1.4202x over previous
import functools

import jax
import jax.numpy as jnp
from jax import lax
from jax.experimental import pallas as pl
from jax.experimental.pallas import tpu as pltpu
from jax.experimental.pallas import tpu_sc as plsc

N = 10000
NP = 10240
E = 320000
D = 128
H = 64
CK = 128
BLK = 8
NB = 20
EP = 16 * NB * BLK * CK
RPT = NP // 16
RC = 64
LANES = 16
GRP = CK // LANES
GLA = 4


def _sc_gcn(x2, epack, ev2):
    mesh = plsc.VectorSubcoreMesh(core_axis_name="c", subcore_axis_name="s")

    @functools.partial(
        pl.kernel,
        out_type=(
            jax.ShapeDtypeStruct((2 * NP, H), jnp.float32),
            jax.ShapeDtypeStruct((2 * NP, H), jnp.float32),
        ),
        mesh=mesh,
        scratch_types=[
            pltpu.VMEM_SHARED((NP, H), jnp.float32),
            pltpu.VMEM((BLK, 2, CK), jnp.int32),
            pltpu.VMEM((BLK, 2, CK), jnp.int32),
            pltpu.VMEM((BLK, CK), jnp.float32),
            pltpu.VMEM((BLK, CK), jnp.float32),
            pltpu.VMEM((BLK, CK), jnp.int32),
        ] + [pltpu.VMEM((CK, H), jnp.float32)] * BLK + [
            pltpu.VMEM((RC, H), jnp.float32),
            pltpu.VMEM((RC, H), jnp.float32),
            pltpu.VMEM((RC, H), jnp.float32),
            pltpu.SemaphoreType.DMA,
            pltpu.SemaphoreType.DMA,
        ] + [pltpu.SemaphoreType.DMA] * BLK
          + [pltpu.SemaphoreType.DMA] * BLK,
        compiler_params=pltpu.CompilerParams(use_tc_tiling_on_sc=False),
    )
    def k(x2_hbm, ep_hbm, ev_hbm, out2_hbm, table_hbm, acc,
          eA, eB, evA, evB, gidx, *rest):
        rbufs = rest[:BLK]
        fa, fb, fc = rest[BLK:BLK + 3]
        esA, esB = rest[BLK + 3:BLK + 5]
        gsems = rest[BLK + 5:2 * BLK + 5]
        ssems = rest[2 * BLK + 5:3 * BLK + 5]
        c = lax.axis_index("c")
        t = lax.axis_index("s")
        coff = c * NP
        row0 = t * RPT

        def zero_acc():
            def zb(r, _):
                for kk in range(H // LANES):
                    fa[r, pl.ds(kk * LANES, LANES)] = jnp.zeros(
                        (LANES,), jnp.float32)
                return 0
            lax.fori_loop(0, RC, zb, 0)
            for j in range(RPT // RC):
                pltpu.sync_copy(fa, acc.at[pl.ds(row0 + j * RC, RC)])

        def start_prefetch(i, eb, evb, es):
            sl = pl.ds((t * NB + i) * BLK, BLK)
            pltpu.async_copy(ep_hbm.at[sl], eb, es)
            pltpu.async_copy(ev_hbm.at[sl], evb, es)

        def wait_prefetch(i, eb, evb, es):
            sl = pl.ds((t * NB + i) * BLK, BLK)
            pltpu.make_async_copy(ep_hbm.at[sl], eb, es).wait()
            pltpu.make_async_copy(ev_hbm.at[sl], evb, es).wait()

        def gather(j):
            pltpu.async_copy(table_hbm.at[gidx.at[j]], rbufs[j], gsems[j])

        def wait_gather(j):
            pltpu.make_async_copy(
                table_hbm.at[gidx.at[j]], rbufs[j], gsems[j]).wait()

        def scatter(j, eb):
            pltpu.async_copy(
                rbufs[j], acc.at[eb.at[j, 0]], ssems[j], add=True)

        def wait_scatter(j, eb):
            pltpu.make_async_copy(
                rbufs[j], acc.at[eb.at[j, 0]], ssems[j]).wait()

        def process_block(eb, evb, not_first):
            for j in range(BLK):
                for g in range(GRP):
                    sl = pl.ds(g * LANES, LANES)
                    gidx[j, sl] = eb[j, 1, sl] + coff
            for j in range(GLA):
                @pl.when(not_first)
                def _(j=j):
                    wait_scatter(j, eb)
                gather(j)
            for j in range(BLK):
                wait_gather(j)
                rj = rbufs[j]

                @plsc.parallel_loop(0, GRP, unroll=2)
                def escale(g):
                    ev16 = evb[j, pl.ds(g * LANES, LANES)]
                    for l in range(LANES):
                        s = jnp.full((LANES,), ev16[l], jnp.float32)
                        e = g * LANES + l
                        for kk in range(H // LANES):
                            sl = pl.ds(kk * LANES, LANES)
                            rj[e, sl] = rj[e, sl] * s
                scatter(j, eb)
                if j + GLA < BLK:
                    @pl.when(not_first)
                    def _(j=j):
                        wait_scatter(j + GLA, eb)
                    gather(j + GLA)

        def layer():
            start_prefetch(0, eA, evA, esA)

            def pair(i, _):
                a = 2 * i
                wait_prefetch(a, eA, evA, esA)
                start_prefetch(a + 1, eB, evB, esB)
                process_block(eA, evA, i > 0)
                wait_prefetch(a + 1, eB, evB, esB)

                @pl.when(i < NB // 2 - 1)
                def _():
                    start_prefetch(a + 2, eA, evA, esA)
                process_block(eB, evB, i >= 0)
                return 0
            lax.fori_loop(0, NB // 2, pair, 0)
            for j in range(BLK):
                wait_scatter(j, eB)

        def copy_rows(src_hbm, dst_hbm):
            for j in range(RPT // RC):
                sl = pl.ds(coff + row0 + j * RC, RC)
                pltpu.sync_copy(src_hbm.at[sl], fb)
                pltpu.sync_copy(fb, dst_hbm.at[sl])

        def flush_acc_to_table():
            for j in range(RPT // RC):
                r0 = row0 + j * RC
                pltpu.sync_copy(acc.at[pl.ds(r0, RC)], fa)
                pltpu.sync_copy(fa, table_hbm.at[pl.ds(coff + r0, RC)])

        copy_rows(x2_hbm, table_hbm)
        zero_acc()
        plsc.subcore_barrier()

        def one_layer(lyr, _):
            layer()
            plsc.subcore_barrier()

            @pl.when(lyr == 0)
            def _():
                flush_acc_to_table()
                zero_acc()
                plsc.subcore_barrier()
            return 0
        lax.fori_loop(0, 2, one_layer, 0)

        third = jnp.float32(1.0 / 3.0)
        for j in range(RPT // RC):
            r0 = row0 + j * RC
            pltpu.sync_copy(acc.at[pl.ds(r0, RC)], fa)
            pltpu.sync_copy(x2_hbm.at[pl.ds(coff + r0, RC)], fb)
            pltpu.sync_copy(table_hbm.at[pl.ds(coff + r0, RC)], fc)

            def mean_row(r, _):
                for kk in range(H // LANES):
                    sl = pl.ds(kk * LANES, LANES)
                    fa[r, sl] = (fa[r, sl] + fb[r, sl] + fc[r, sl]) * third
                return 0
            lax.fori_loop(0, RC, mean_row, 0)
            pltpu.sync_copy(fa, out2_hbm.at[pl.ds(coff + r0, RC)])

    return k(x2, epack, ev2)


def kernel(x, edge_index, edge_values, keep_rate):
    del keep_rate
    pad = EP - E
    src = jnp.concatenate([edge_index[0], jnp.zeros((pad,), jnp.int32)])
    dst = jnp.concatenate([edge_index[1], jnp.zeros((pad,), jnp.int32)])
    ev = jnp.concatenate([edge_values, jnp.zeros((pad,), jnp.float32)])
    epack = jnp.stack(
        [src.reshape(EP // CK, CK), dst.reshape(EP // CK, CK)], axis=1)
    ev2 = ev.reshape(EP // CK, CK)
    x2 = x.reshape(N, 2, H).transpose(1, 0, 2)
    x2 = jnp.pad(x2, ((0, 0), (0, NP - N), (0, 0))).reshape(2 * NP, H)
    out2, _ = _sc_gcn(x2, epack, ev2)
    return out2.reshape(2, NP, H)[:, :N].transpose(1, 0, 2).reshape(N, D)


# speedup vs baseline: 4.6638x; 1.0020x over previous
import functools

import jax
import jax.numpy as jnp
from jax import lax
from jax.experimental import pallas as pl
from jax.experimental.pallas import tpu as pltpu
from jax.experimental.pallas import tpu_sc as plsc

N = 10000
NP = 10240
E = 320000
D = 128
H = 64
CK = 128
BLK = 8
NB = 20
EP = 16 * NB * BLK * CK
RPT = NP // 16
RC = 64
LANES = 16
GRP = CK // LANES
GLA = 4


def _sc_gcn(x2, epack, ev2):
    mesh = plsc.VectorSubcoreMesh(core_axis_name="c", subcore_axis_name="s")

    @functools.partial(
        pl.kernel,
        out_type=(
            jax.ShapeDtypeStruct((2 * NP, H), jnp.float32),
            jax.ShapeDtypeStruct((2 * NP, H), jnp.float32),
        ),
        mesh=mesh,
        scratch_types=[
            pltpu.VMEM_SHARED((NP, H), jnp.float32),
            pltpu.VMEM((BLK, 2, CK), jnp.int32),
            pltpu.VMEM((BLK, 2, CK), jnp.int32),
            pltpu.VMEM((BLK, CK), jnp.float32),
            pltpu.VMEM((BLK, CK), jnp.float32),
            pltpu.VMEM((BLK, CK), jnp.int32),
            pltpu.VMEM((BLK, CK), jnp.int32),
        ] + [pltpu.VMEM((CK, H), jnp.float32)] * BLK + [
            pltpu.VMEM((RC, H), jnp.float32),
            pltpu.VMEM((RC, H), jnp.float32),
            pltpu.VMEM((RC, H), jnp.float32),
            pltpu.SemaphoreType.DMA,
            pltpu.SemaphoreType.DMA,
        ] + [pltpu.SemaphoreType.DMA] * BLK
          + [pltpu.SemaphoreType.DMA] * BLK,
        compiler_params=pltpu.CompilerParams(use_tc_tiling_on_sc=False),
    )
    def k(x2_hbm, ep_hbm, ev_hbm, out2_hbm, table_hbm, acc,
          eA, eB, evA, evB, gidx, sidx, *rest):
        rbufs = rest[:BLK]
        fa, fb, fc = rest[BLK:BLK + 3]
        esA, esB = rest[BLK + 3:BLK + 5]
        gsems = rest[BLK + 5:2 * BLK + 5]
        ssems = rest[2 * BLK + 5:3 * BLK + 5]
        c = lax.axis_index("c")
        t = lax.axis_index("s")
        coff = c * NP
        row0 = t * RPT

        def zero_acc():
            def zb(r, _):
                for kk in range(H // LANES):
                    fa[r, pl.ds(kk * LANES, LANES)] = jnp.zeros(
                        (LANES,), jnp.float32)
                return 0
            lax.fori_loop(0, RC, zb, 0)
            for j in range(RPT // RC):
                pltpu.sync_copy(fa, acc.at[pl.ds(row0 + j * RC, RC)])

        def start_prefetch(i, eb, evb, es):
            sl = pl.ds((t * NB + i) * BLK, BLK)
            pltpu.async_copy(ep_hbm.at[sl], eb, es)
            pltpu.async_copy(ev_hbm.at[sl], evb, es)

        def wait_prefetch(i, eb, evb, es):
            sl = pl.ds((t * NB + i) * BLK, BLK)
            pltpu.make_async_copy(ep_hbm.at[sl], eb, es).wait()
            pltpu.make_async_copy(ev_hbm.at[sl], evb, es).wait()

        def gather(j):
            pltpu.async_copy(table_hbm.at[gidx.at[j]], rbufs[j], gsems[j])

        def wait_gather(j):
            pltpu.make_async_copy(
                table_hbm.at[gidx.at[j]], rbufs[j], gsems[j]).wait()

        def scatter(j):
            pltpu.async_copy(
                rbufs[j], acc.at[sidx.at[j]], ssems[j], add=True)

        def wait_scatter(j):
            pltpu.make_async_copy(
                rbufs[j], acc.at[sidx.at[j]], ssems[j]).wait()

        def process_block(eb, evb, not_first):
            for j in range(BLK):
                for g in range(GRP):
                    sl = pl.ds(g * LANES, LANES)
                    gidx[j, sl] = eb[j, 1, sl] + coff
            for j in range(GLA):
                @pl.when(not_first)
                def _(j=j):
                    wait_scatter(j)
                for g in range(GRP):
                    sl = pl.ds(g * LANES, LANES)
                    sidx[j, sl] = eb[j, 0, sl]
                gather(j)
            for j in range(BLK):
                wait_gather(j)
                rj = rbufs[j]

                @plsc.parallel_loop(0, GRP, unroll=2)
                def escale(g):
                    ev16 = evb[j, pl.ds(g * LANES, LANES)]
                    for l in range(LANES):
                        s = jnp.full((LANES,), ev16[l], jnp.float32)
                        e = g * LANES + l
                        for kk in range(H // LANES):
                            sl = pl.ds(kk * LANES, LANES)
                            rj[e, sl] = rj[e, sl] * s
                scatter(j)
                if j + GLA < BLK:
                    @pl.when(not_first)
                    def _(j=j):
                        wait_scatter(j + GLA)
                    for g in range(GRP):
                        sl = pl.ds(g * LANES, LANES)
                        sidx[j + GLA, sl] = eb[j + GLA, 0, sl]
                    gather(j + GLA)

        def layer():
            start_prefetch(0, eA, evA, esA)

            def pair(i, _):
                a = 2 * i
                wait_prefetch(a, eA, evA, esA)
                start_prefetch(a + 1, eB, evB, esB)
                process_block(eA, evA, i > 0)
                wait_prefetch(a + 1, eB, evB, esB)

                @pl.when(i < NB // 2 - 1)
                def _():
                    start_prefetch(a + 2, eA, evA, esA)
                process_block(eB, evB, i >= 0)
                return 0
            lax.fori_loop(0, NB // 2, pair, 0)
            for j in range(BLK):
                wait_scatter(j)

        def copy_rows(src_hbm, dst_hbm):
            for j in range(RPT // RC):
                sl = pl.ds(coff + row0 + j * RC, RC)
                pltpu.sync_copy(src_hbm.at[sl], fb)
                pltpu.sync_copy(fb, dst_hbm.at[sl])

        def flush_acc_to_table():
            for j in range(RPT // RC):
                r0 = row0 + j * RC
                pltpu.sync_copy(acc.at[pl.ds(r0, RC)], fa)
                pltpu.sync_copy(fa, table_hbm.at[pl.ds(coff + r0, RC)])

        copy_rows(x2_hbm, table_hbm)
        zero_acc()
        plsc.subcore_barrier()

        def one_layer(lyr, _):
            layer()
            plsc.subcore_barrier()

            @pl.when(lyr == 0)
            def _():
                flush_acc_to_table()
                zero_acc()
                plsc.subcore_barrier()
            return 0
        lax.fori_loop(0, 2, one_layer, 0)

        third = jnp.float32(1.0 / 3.0)
        for j in range(RPT // RC):
            r0 = row0 + j * RC
            pltpu.sync_copy(acc.at[pl.ds(r0, RC)], fa)
            pltpu.sync_copy(x2_hbm.at[pl.ds(coff + r0, RC)], fb)
            pltpu.sync_copy(table_hbm.at[pl.ds(coff + r0, RC)], fc)

            def mean_row(r, _):
                for kk in range(H // LANES):
                    sl = pl.ds(kk * LANES, LANES)
                    fa[r, sl] = (fa[r, sl] + fb[r, sl] + fc[r, sl]) * third
                return 0
            lax.fori_loop(0, RC, mean_row, 0)
            pltpu.sync_copy(fa, out2_hbm.at[pl.ds(coff + r0, RC)])

    return k(x2, epack, ev2)


def kernel(x, edge_index, edge_values, keep_rate):
    del keep_rate
    pad = EP - E
    src = jnp.concatenate([edge_index[0], jnp.zeros((pad,), jnp.int32)])
    dst = jnp.concatenate([edge_index[1], jnp.zeros((pad,), jnp.int32)])
    ev = jnp.concatenate([edge_values, jnp.zeros((pad,), jnp.float32)])
    epack = jnp.stack(
        [src.reshape(EP // CK, CK), dst.reshape(EP // CK, CK)], axis=1)
    ev2 = ev.reshape(EP // CK, CK)
    x2 = x.reshape(N, 2, H).transpose(1, 0, 2)
    x2 = jnp.pad(x2, ((0, 0), (0, NP - N), (0, 0))).reshape(2 * NP, H)
    out2, _ = _sc_gcn(x2, epack, ev2)
    return out2.reshape(2, NP, H)[:, :N].transpose(1, 0, 2).reshape(N, D)


# speedup vs baseline: 6.1771x; 1.3245x over previous
import functools

import jax
import jax.numpy as jnp
from jax import lax
from jax.experimental import pallas as pl
from jax.experimental.pallas import tpu as pltpu
from jax.experimental.pallas import tpu_sc as plsc

N = 10000
NP = 10240
E = 320000
D = 128
H = 64
CK = 128
BLK = 8
NSC = 4
NB = 20
EP = 16 * NB * BLK * CK
RPT = NP // 16
RC = 64
LANES = 16
GRP = CK // LANES
GLA = 4
ILV = plsc.PackFormat.INTERLEAVED
_DNUMS = lax.GatherDimensionNumbers(
    offset_dims=(), collapsed_slice_dims=(0,), start_index_map=(0,))


def _sc_layer(table_b, epack, ev2, xmean, cmean, final):
    mesh = plsc.VectorSubcoreMesh(core_axis_name="c", subcore_axis_name="s")

    @functools.partial(
        pl.kernel,
        out_type=jax.ShapeDtypeStruct((2 * NP, H), jnp.float32),
        mesh=mesh,
        scratch_types=[
            pltpu.VMEM_SHARED((NP, H), jnp.float32),
            pltpu.VMEM((BLK, 2, CK), jnp.int32),
            pltpu.VMEM((BLK, 2, CK), jnp.int32),
            pltpu.VMEM((BLK, CK), jnp.float32),
            pltpu.VMEM((BLK, CK), jnp.float32),
            pltpu.VMEM((BLK, CK), jnp.int32),
            pltpu.VMEM((BLK, CK), jnp.int32),
        ] + [pltpu.VMEM((CK, H), jnp.bfloat16)] * BLK
          + [pltpu.VMEM((CK, H), jnp.float32)] * NSC + [
            pltpu.VMEM((RC, H), jnp.float32),
            pltpu.VMEM((RC, H), jnp.float32),
            pltpu.VMEM((RC, H), jnp.float32),
            pltpu.SemaphoreType.DMA,
            pltpu.SemaphoreType.DMA,
        ] + [pltpu.SemaphoreType.DMA] * BLK
          + [pltpu.SemaphoreType.DMA] * NSC,
        compiler_params=pltpu.CompilerParams(
            use_tc_tiling_on_sc=False, needs_layout_passes=False),
    )
    def k(table_hbm, ep_hbm, ev_hbm, xm_hbm, cm_hbm, out_hbm, acc,
          eA, eB, evA, evB, gidx, sidx, *rest):
        rbufs = rest[:BLK]
        sbufs = rest[BLK:BLK + NSC]
        fa, fb, fc = rest[BLK + NSC:BLK + NSC + 3]
        esA, esB = rest[BLK + NSC + 3:BLK + NSC + 5]
        gsems = rest[BLK + NSC + 5:2 * BLK + NSC + 5]
        ssems = rest[2 * BLK + NSC + 5:2 * BLK + 2 * NSC + 5]
        c = lax.axis_index("c")
        t = lax.axis_index("s")
        coff = c * NP
        row0 = t * RPT

        def zero_acc():
            def zb(r, _):
                for kk in range(H // LANES):
                    fa[r, pl.ds(kk * LANES, LANES)] = jnp.zeros(
                        (LANES,), jnp.float32)
                return 0
            lax.fori_loop(0, RC, zb, 0)
            for j in range(RPT // RC):
                pltpu.sync_copy(fa, acc.at[pl.ds(row0 + j * RC, RC)])

        def start_prefetch(i, eb, evb, es):
            sl = pl.ds((t * NB + i) * BLK, BLK)
            pltpu.async_copy(ep_hbm.at[sl], eb, es)
            pltpu.async_copy(ev_hbm.at[sl], evb, es)

        def wait_prefetch(i, eb, evb, es):
            sl = pl.ds((t * NB + i) * BLK, BLK)
            pltpu.make_async_copy(ep_hbm.at[sl], eb, es).wait()
            pltpu.make_async_copy(ev_hbm.at[sl], evb, es).wait()

        def gather(j):
            pltpu.async_copy(table_hbm.at[gidx.at[j]], rbufs[j], gsems[j])

        def wait_gather(j):
            pltpu.make_async_copy(
                table_hbm.at[gidx.at[j]], rbufs[j], gsems[j]).wait()

        def scatter(j):
            pltpu.async_copy(
                sbufs[j % NSC], acc.at[sidx.at[j]], ssems[j % NSC], add=True)

        def wait_scatter(j):
            pltpu.make_async_copy(
                sbufs[j % NSC], acc.at[sidx.at[j]], ssems[j % NSC]).wait()

        def process_block(eb, evb, not_first):
            for j in range(BLK):
                for g in range(GRP):
                    sl = pl.ds(g * LANES, LANES)
                    gidx[j, sl] = eb[j, 1, sl] + coff
            for j in range(GLA):
                for g in range(GRP):
                    sl = pl.ds(g * LANES, LANES)
                    sidx[j, sl] = eb[j, 0, sl]
                gather(j)
            for j in range(BLK):
                wait_gather(j)
                if j >= NSC:
                    wait_scatter(j - NSC)
                else:
                    @pl.when(not_first)
                    def _(j=j):
                        wait_scatter(j + BLK - NSC)
                rj = rbufs[j]
                sj = sbufs[j % NSC]

                @plsc.parallel_loop(0, CK, unroll=2)
                def escale(e):
                    ev16 = evb[j, pl.ds((e >> 4) << 4, LANES)]
                    lidx = jnp.full((LANES,), e & (LANES - 1), jnp.int32)
                    sv = lax.gather(
                        ev16, lidx[:, None], _DNUMS, (1,),
                        mode=lax.GatherScatterMode.PROMISE_IN_BOUNDS)
                    for h2 in range(H // 32):
                        v32 = rj[e, pl.ds(32 * h2, 32)]
                        a, b = plsc.unpack(
                            v32, format=ILV,
                            preferred_element_type=jnp.float32)
                        sj[e, pl.ds(32 * h2, LANES)] = a * sv
                        sj[e, pl.ds(32 * h2 + LANES, LANES)] = b * sv
                scatter(j)
                if j + GLA < BLK:
                    for g in range(GRP):
                        sl = pl.ds(g * LANES, LANES)
                        sidx[j + GLA, sl] = eb[j + GLA, 0, sl]
                    gather(j + GLA)

        zero_acc()
        plsc.subcore_barrier()

        start_prefetch(0, eA, evA, esA)

        def pair(i, _):
            a = 2 * i
            wait_prefetch(a, eA, evA, esA)
            start_prefetch(a + 1, eB, evB, esB)
            process_block(eA, evA, i > 0)
            wait_prefetch(a + 1, eB, evB, esB)

            @pl.when(i < NB // 2 - 1)
            def _():
                start_prefetch(a + 2, eA, evA, esA)
            process_block(eB, evB, i >= 0)
            return 0
        lax.fori_loop(0, NB // 2, pair, 0)
        for j in range(BLK - NSC, BLK):
            wait_scatter(j)
        plsc.subcore_barrier()

        if not final:
            for j in range(RPT // RC):
                r0 = row0 + j * RC
                pltpu.sync_copy(acc.at[pl.ds(r0, RC)], fa)
                pltpu.sync_copy(fa, out_hbm.at[pl.ds(coff + r0, RC)])
        else:
            third = jnp.float32(1.0 / 3.0)
            for j in range(RPT // RC):
                r0 = row0 + j * RC
                pltpu.sync_copy(acc.at[pl.ds(r0, RC)], fa)
                pltpu.sync_copy(xm_hbm.at[pl.ds(coff + r0, RC)], fb)
                pltpu.sync_copy(cm_hbm.at[pl.ds(coff + r0, RC)], fc)

                def mean_row(r, _):
                    for kk in range(H // LANES):
                        sl = pl.ds(kk * LANES, LANES)
                        fa[r, sl] = (fa[r, sl] + fb[r, sl] + fc[r, sl]) * third
                    return 0
                lax.fori_loop(0, RC, mean_row, 0)
                pltpu.sync_copy(fa, out_hbm.at[pl.ds(coff + r0, RC)])

    return k(table_b, epack, ev2, xmean, cmean)


def _perm(a):
    s = a.shape[:-1]
    return a.reshape(*s, H // 32, 16, 2).swapaxes(-1, -2).reshape(*s, H)


def _unperm(a):
    s = a.shape[:-1]
    return a.reshape(*s, H // 32, 2, 16).swapaxes(-1, -2).reshape(*s, H)


def kernel(x, edge_index, edge_values, keep_rate):
    del keep_rate
    pad = EP - E
    src = jnp.concatenate([edge_index[0], jnp.zeros((pad,), jnp.int32)])
    dst = jnp.concatenate([edge_index[1], jnp.zeros((pad,), jnp.int32)])
    ev = jnp.concatenate([edge_values, jnp.zeros((pad,), jnp.float32)])
    epack = jnp.stack(
        [src.reshape(EP // CK, CK), dst.reshape(EP // CK, CK)], axis=1)
    ev2 = ev.reshape(EP // CK, CK)
    x2 = x.reshape(N, 2, H).transpose(1, 0, 2)
    x2 = jnp.pad(x2, ((0, 0), (0, NP - N), (0, 0))).reshape(2 * NP, H)

    cur1_p = _sc_layer(x2.astype(jnp.bfloat16), epack, ev2, x2, x2, False)
    cur1_b = _unperm(cur1_p).astype(jnp.bfloat16)
    out_p = _sc_layer(cur1_b, epack, ev2, _perm(x2), cur1_p, True)
    out2 = _unperm(out_p)
    return out2.reshape(2, NP, H)[:, :N].transpose(1, 0, 2).reshape(N, D)


# speedup vs baseline: 6.1786x; 1.0002x over previous
import functools

import jax
import jax.numpy as jnp
from jax import lax
from jax.experimental import pallas as pl
from jax.experimental.pallas import tpu as pltpu
from jax.experimental.pallas import tpu_sc as plsc

N = 10000
NP = 10240
E = 320000
D = 128
H = 64
CK = 128
BLK = 8
NSC = 4
NB = 20
EP = 16 * NB * BLK * CK
RPT = NP // 16
RC = 64
LANES = 16
GRP = CK // LANES
GLA = 4
ILV = plsc.PackFormat.INTERLEAVED
_DNUMS = lax.GatherDimensionNumbers(
    offset_dims=(), collapsed_slice_dims=(0,), start_index_map=(0,))


def _sc_layer(table_b, epack, ev2, xmean, cmean, final):
    mesh = plsc.VectorSubcoreMesh(core_axis_name="c", subcore_axis_name="s")

    @functools.partial(
        pl.kernel,
        out_type=jax.ShapeDtypeStruct((2 * NP, H), jnp.float32),
        mesh=mesh,
        scratch_types=[
            pltpu.VMEM_SHARED((NP, H), jnp.float32),
            pltpu.VMEM((BLK, 2, CK), jnp.int32),
            pltpu.VMEM((BLK, 2, CK), jnp.int32),
            pltpu.VMEM((BLK, CK), jnp.float32),
            pltpu.VMEM((BLK, CK), jnp.float32),
            pltpu.VMEM((BLK, CK), jnp.int32),
            pltpu.VMEM((BLK, CK), jnp.int32),
        ] + [pltpu.VMEM((CK, H), jnp.bfloat16)] * BLK
          + [pltpu.VMEM((CK, H), jnp.float32)] * NSC + [
            pltpu.VMEM((RC, H), jnp.float32),
            pltpu.VMEM((RC, H), jnp.float32),
            pltpu.VMEM((RC, H), jnp.float32),
            pltpu.SemaphoreType.DMA,
            pltpu.SemaphoreType.DMA,
        ] + [pltpu.SemaphoreType.DMA] * BLK
          + [pltpu.SemaphoreType.DMA] * NSC,
        compiler_params=pltpu.CompilerParams(
            use_tc_tiling_on_sc=False, needs_layout_passes=False),
    )
    def k(table_hbm, ep_hbm, ev_hbm, xm_hbm, cm_hbm, out_hbm, acc,
          eA, eB, evA, evB, gidx, sidx, *rest):
        rbufs = rest[:BLK]
        sbufs = rest[BLK:BLK + NSC]
        fa, fb, fc = rest[BLK + NSC:BLK + NSC + 3]
        esA, esB = rest[BLK + NSC + 3:BLK + NSC + 5]
        gsems = rest[BLK + NSC + 5:2 * BLK + NSC + 5]
        ssems = rest[2 * BLK + NSC + 5:2 * BLK + 2 * NSC + 5]
        c = lax.axis_index("c")
        t = lax.axis_index("s")
        coff = c * NP
        row0 = t * RPT

        def zero_acc():
            def zb(r, _):
                for kk in range(H // LANES):
                    fa[r, pl.ds(kk * LANES, LANES)] = jnp.zeros(
                        (LANES,), jnp.float32)
                return 0
            lax.fori_loop(0, RC, zb, 0)
            for j in range(RPT // RC):
                pltpu.sync_copy(fa, acc.at[pl.ds(row0 + j * RC, RC)])

        def start_prefetch(i, eb, evb, es):
            sl = pl.ds((t * NB + i) * BLK, BLK)
            pltpu.async_copy(ep_hbm.at[sl], eb, es)
            pltpu.async_copy(ev_hbm.at[sl], evb, es)

        def wait_prefetch(i, eb, evb, es):
            sl = pl.ds((t * NB + i) * BLK, BLK)
            pltpu.make_async_copy(ep_hbm.at[sl], eb, es).wait()
            pltpu.make_async_copy(ev_hbm.at[sl], evb, es).wait()

        def gather(j):
            pltpu.async_copy(table_hbm.at[gidx.at[j]], rbufs[j], gsems[j])

        def wait_gather(j):
            pltpu.make_async_copy(
                table_hbm.at[gidx.at[j]], rbufs[j], gsems[j]).wait()

        def scatter(j):
            pltpu.async_copy(
                sbufs[j % NSC], acc.at[sidx.at[j]], ssems[j % NSC], add=True)

        def wait_scatter(j):
            pltpu.make_async_copy(
                sbufs[j % NSC], acc.at[sidx.at[j]], ssems[j % NSC]).wait()

        def process_block(eb, evb, not_first):
            for j in range(BLK):
                for g in range(GRP):
                    sl = pl.ds(g * LANES, LANES)
                    gidx[j, sl] = eb[j, 1, sl] + coff
            for j in range(GLA):
                for g in range(GRP):
                    sl = pl.ds(g * LANES, LANES)
                    sidx[j, sl] = eb[j, 0, sl]
                gather(j)
            for j in range(BLK):
                wait_gather(j)
                if j >= NSC:
                    wait_scatter(j - NSC)
                else:
                    @pl.when(not_first)
                    def _(j=j):
                        wait_scatter(j + BLK - NSC)
                rj = rbufs[j]
                sj = sbufs[j % NSC]

                @plsc.parallel_loop(0, CK, unroll=4)
                def escale(e):
                    ev16 = evb[j, pl.ds((e >> 4) << 4, LANES)]
                    lidx = jnp.full((LANES,), e & (LANES - 1), jnp.int32)
                    sv = lax.gather(
                        ev16, lidx[:, None], _DNUMS, (1,),
                        mode=lax.GatherScatterMode.PROMISE_IN_BOUNDS)
                    for h2 in range(H // 32):
                        v32 = rj[e, pl.ds(32 * h2, 32)]
                        a, b = plsc.unpack(
                            v32, format=ILV,
                            preferred_element_type=jnp.float32)
                        sj[e, pl.ds(32 * h2, LANES)] = a * sv
                        sj[e, pl.ds(32 * h2 + LANES, LANES)] = b * sv
                scatter(j)
                if j + GLA < BLK:
                    for g in range(GRP):
                        sl = pl.ds(g * LANES, LANES)
                        sidx[j + GLA, sl] = eb[j + GLA, 0, sl]
                    gather(j + GLA)

        zero_acc()
        plsc.subcore_barrier()

        start_prefetch(0, eA, evA, esA)

        def pair(i, _):
            a = 2 * i
            wait_prefetch(a, eA, evA, esA)
            start_prefetch(a + 1, eB, evB, esB)
            process_block(eA, evA, i > 0)
            wait_prefetch(a + 1, eB, evB, esB)

            @pl.when(i < NB // 2 - 1)
            def _():
                start_prefetch(a + 2, eA, evA, esA)
            process_block(eB, evB, i >= 0)
            return 0
        lax.fori_loop(0, NB // 2, pair, 0)
        for j in range(BLK - NSC, BLK):
            wait_scatter(j)
        plsc.subcore_barrier()

        if not final:
            for j in range(RPT // RC):
                r0 = row0 + j * RC
                pltpu.sync_copy(acc.at[pl.ds(r0, RC)], fa)
                pltpu.sync_copy(fa, out_hbm.at[pl.ds(coff + r0, RC)])
        else:
            third = jnp.float32(1.0 / 3.0)
            for j in range(RPT // RC):
                r0 = row0 + j * RC
                pltpu.sync_copy(acc.at[pl.ds(r0, RC)], fa)
                pltpu.sync_copy(xm_hbm.at[pl.ds(coff + r0, RC)], fb)
                pltpu.sync_copy(cm_hbm.at[pl.ds(coff + r0, RC)], fc)

                def mean_row(r, _):
                    for kk in range(H // LANES):
                        sl = pl.ds(kk * LANES, LANES)
                        fa[r, sl] = (fa[r, sl] + fb[r, sl] + fc[r, sl]) * third
                    return 0
                lax.fori_loop(0, RC, mean_row, 0)
                pltpu.sync_copy(fa, out_hbm.at[pl.ds(coff + r0, RC)])

    return k(table_b, epack, ev2, xmean, cmean)


def _perm(a):
    s = a.shape[:-1]
    return a.reshape(*s, H // 32, 16, 2).swapaxes(-1, -2).reshape(*s, H)


def _unperm(a):
    s = a.shape[:-1]
    return a.reshape(*s, H // 32, 2, 16).swapaxes(-1, -2).reshape(*s, H)


def kernel(x, edge_index, edge_values, keep_rate):
    del keep_rate
    pad = EP - E
    src = jnp.concatenate([edge_index[0], jnp.zeros((pad,), jnp.int32)])
    dst = jnp.concatenate([edge_index[1], jnp.zeros((pad,), jnp.int32)])
    ev = jnp.concatenate([edge_values, jnp.zeros((pad,), jnp.float32)])
    epack = jnp.stack(
        [src.reshape(EP // CK, CK), dst.reshape(EP // CK, CK)], axis=1)
    ev2 = ev.reshape(EP // CK, CK)
    x2 = x.reshape(N, 2, H).transpose(1, 0, 2)
    x2 = jnp.pad(x2, ((0, 0), (0, NP - N), (0, 0))).reshape(2 * NP, H)

    cur1_p = _sc_layer(x2.astype(jnp.bfloat16), epack, ev2, x2, x2, False)
    cur1_b = _unperm(cur1_p).astype(jnp.bfloat16)
    out_p = _sc_layer(cur1_b, epack, ev2, _perm(x2), cur1_p, True)
    out2 = _unperm(out_p)
    return out2.reshape(2, NP, H)[:, :N].transpose(1, 0, 2).reshape(N, D)
